# baseline (speedup 1.0000x reference)
import jax
import jax.numpy as jnp
from jax import lax
from jax.experimental import pallas as pl
from jax.experimental.pallas import tpu as pltpu
from jax.experimental.pallas import tpu_sc as plsc

B = 4096
V = 100000
NC = 2
NS = 16
L = 16
NW = NC * NS
BPW = B // NW
NV = BPW // L


def _sc_body(inp_hbm, tgt_hbm, out_hbm, tgt_v, idx_v, val_v, acc_v, sem):
    wid = lax.axis_index("s") * NC + lax.axis_index("c")
    base = wid * BPW
    pltpu.sync_copy(tgt_hbm.at[pl.ds(base, BPW)], tgt_v)
    lane = lax.iota(jnp.int32, 16)
    for i in range(NV):
        t = tgt_v[pl.ds(i * L, L)]
        row = base + i * L + lane
        idx_v[pl.ds(i * L, L)] = row * V + t
    pltpu.async_copy(inp_hbm.at[idx_v], val_v, sem).wait()
    acc = jnp.zeros((L,), jnp.float32)
    for i in range(NV):
        t = tgt_v[pl.ds(i * L, L)]
        v = val_v[pl.ds(i * L, L)]
        acc = acc + jnp.where(t > 0, v, jnp.float32(0.0))
    acc_v[...] = acc
    pltpu.sync_copy(acc_v, out_hbm.at[wid])


@jax.jit
def kernel(input, target):
    flat = input.reshape(B * V)
    tgt = target.reshape(B).astype(jnp.int32)
    mesh = plsc.VectorSubcoreMesh(core_axis_name="c", subcore_axis_name="s")
    parts = pl.kernel(
        _sc_body,
        out_type=jax.ShapeDtypeStruct((NW, L), jnp.float32),
        mesh=mesh,
        scratch_types=[
            pltpu.VMEM((BPW,), jnp.int32),
            pltpu.VMEM((BPW,), jnp.int32),
            pltpu.VMEM((BPW,), jnp.float32),
            pltpu.VMEM((L,), jnp.float32),
            pltpu.SemaphoreType.DMA,
        ],
    )(flat, tgt)
    return -jnp.sum(parts)


# speedup vs baseline: 2.3882x; 2.3882x over previous
import jax
import jax.numpy as jnp
from jax import lax
from jax.experimental import pallas as pl
from jax.experimental.pallas import tpu as pltpu
from jax.experimental.pallas import tpu_sc as plsc

B = 4096
V = 100000
NC = 2
NS = 16
L = 16
NW = NC * NS
BPW = B // NW
NV = BPW // L
NP = 2
VPP = NV // NP
CPT = VPP * L


def _sc_body(inp_hbm, tgt_hbm, out_hbm, tgt_v, tile_v, acc_v, sem):
    wid = lax.axis_index("s") * NC + lax.axis_index("c")
    base = pl.multiple_of(wid * BPW, BPW)
    pltpu.sync_copy(tgt_hbm.at[pl.ds(base, BPW)], tgt_v)
    lane_iota = lax.iota(jnp.int32, L)
    subrow = lane_iota & 7
    acc = jnp.zeros((L,), jnp.float32)
    for p in range(NP):
        handles = []
        for i in range(VPP):
            ii = p * VPP + i
            t16 = tgt_v[pl.ds(ii * L, L)]
            for j in range(L):
                c0 = pl.multiple_of((t16[j] >> 7) << 7, 128)
                r0 = base + ii * L + (j & ~7)
                handles.append(
                    pltpu.async_copy(
                        inp_hbm.at[pl.ds(r0, 8), pl.ds(c0, 128)],
                        tile_v.at[i * L + j],
                        sem,
                    )
                )
        for h in handles:
            h.wait()
        for i in range(VPP):
            ii = p * VPP + i
            t16 = tgt_v[pl.ds(ii * L, L)]
            tid = lane_iota + i * L
            vals = plsc.load_gather(tile_v, [tid, subrow, t16 & 127])
            acc = acc + jnp.where(t16 > 0, vals, jnp.float32(0.0))
    acc_v[...] = acc
    pltpu.sync_copy(acc_v, out_hbm.at[wid])


@jax.jit
def kernel(input, target):
    tgt = target.reshape(B).astype(jnp.int32)
    mesh = plsc.VectorSubcoreMesh(core_axis_name="c", subcore_axis_name="s")
    parts = pl.kernel(
        _sc_body,
        out_type=jax.ShapeDtypeStruct((NW, L), jnp.float32),
        mesh=mesh,
        compiler_params=pltpu.CompilerParams(needs_layout_passes=False),
        scratch_types=[
            pltpu.VMEM((BPW,), jnp.int32),
            pltpu.VMEM((CPT, 8, 128), jnp.float32),
            pltpu.VMEM((L,), jnp.float32),
            pltpu.SemaphoreType.DMA,
        ],
    )(input, tgt)
    return -jnp.sum(parts)


# speedup vs baseline: 106.5915x; 44.6331x over previous
import jax
import jax.numpy as jnp
from jax import lax
from jax.experimental import pallas as pl
from jax.experimental.pallas import tpu as pltpu
from jax.experimental.pallas import tpu_sc as plsc

B = 4096
V = 100000
NC = 2
NS = 16
L = 16
NW = NC * NS
BPW = B // NW
NV = BPW // L
NP = 2
VPP = NV // NP
CPT = VPP * L


def _sc_body(inpt_hbm, tgt_hbm, out_hbm, tgt_v, tile_v, acc_v, sem):
    wid = lax.axis_index("s") * NC + lax.axis_index("c")
    base = pl.multiple_of(wid * BPW, BPW)
    pltpu.sync_copy(tgt_hbm.at[pl.ds(base, BPW)], tgt_v)
    lane_iota = lax.iota(jnp.int32, L)
    acc = jnp.zeros((L,), jnp.float32)
    for p in range(NP):
        handles = []
        for i in range(VPP):
            ii = p * VPP + i
            t16 = tgt_v[pl.ds(ii * L, L)]
            for j in range(L):
                t0 = pl.multiple_of((t16[j] >> 3) << 3, 8)
                handles.append(
                    pltpu.async_copy(
                        inpt_hbm.at[pl.ds(t0, 8), pl.ds(base, 128)],
                        tile_v.at[i * L + j],
                        sem,
                    )
                )
        for h in handles:
            h.wait()
        for i in range(VPP):
            ii = p * VPP + i
            t16 = tgt_v[pl.ds(ii * L, L)]
            tid = lane_iota + i * L
            col = lane_iota + ii * L
            vals = plsc.load_gather(tile_v, [tid, t16 & 7, col])
            acc = acc + jnp.where(t16 > 0, vals, jnp.float32(0.0))
    acc_v[...] = acc
    pltpu.sync_copy(acc_v, out_hbm.at[wid])


@jax.jit
def kernel(input, target):
    tgt = target.reshape(B).astype(jnp.int32)
    mesh = plsc.VectorSubcoreMesh(core_axis_name="c", subcore_axis_name="s")
    parts = pl.kernel(
        _sc_body,
        out_type=jax.ShapeDtypeStruct((NW, L), jnp.float32),
        mesh=mesh,
        compiler_params=pltpu.CompilerParams(needs_layout_passes=False),
        scratch_types=[
            pltpu.VMEM((BPW,), jnp.int32),
            pltpu.VMEM((CPT, 8, 128), jnp.float32),
            pltpu.VMEM((L,), jnp.float32),
            pltpu.SemaphoreType.DMA,
        ],
    )(input.T, tgt)
    return -jnp.sum(parts)


# speedup vs baseline: 138.6652x; 1.3009x over previous
import jax
import jax.numpy as jnp
from jax import lax
from jax.experimental import pallas as pl
from jax.experimental.pallas import tpu as pltpu
from jax.experimental.pallas import tpu_sc as plsc

B = 4096
V = 100000
NC = 2
NS = 16
L = 16
NW = NC * NS
BPW = B // NW
NV = BPW // L


def _sc_body(inpt_hbm, tgt_hbm, out_hbm, tgt_v, seg_v, acc_v, sem):
    wid = lax.axis_index("s") * NC + lax.axis_index("c")
    base = pl.multiple_of(wid * BPW, BPW)
    pltpu.sync_copy(tgt_hbm.at[pl.ds(base, BPW)], tgt_v)
    pltpu.async_copy(
        inpt_hbm.at[tgt_v, pl.ds(base, BPW)], seg_v, sem
    ).wait()
    lane_iota = lax.iota(jnp.int32, L)
    acc = jnp.zeros((L,), jnp.float32)
    for i in range(NV):
        t16 = tgt_v[pl.ds(i * L, L)]
        diag = lane_iota + i * L
        vals = plsc.load_gather(seg_v, [diag, diag])
        acc = acc + jnp.where(t16 > 0, vals, jnp.float32(0.0))
    acc_v[...] = acc
    pltpu.sync_copy(acc_v, out_hbm.at[wid])


@jax.jit
def kernel(input, target):
    tgt = target.reshape(B).astype(jnp.int32)
    mesh = plsc.VectorSubcoreMesh(core_axis_name="c", subcore_axis_name="s")
    parts = pl.kernel(
        _sc_body,
        out_type=jax.ShapeDtypeStruct((NW, L), jnp.float32),
        mesh=mesh,
        compiler_params=pltpu.CompilerParams(needs_layout_passes=False),
        scratch_types=[
            pltpu.VMEM((BPW,), jnp.int32),
            pltpu.VMEM((BPW, BPW), jnp.float32),
            pltpu.VMEM((L,), jnp.float32),
            pltpu.SemaphoreType.DMA,
        ],
    )(input.T, tgt)
    return -jnp.sum(parts)
